# baseline (speedup 1.0000x reference)
import functools

import jax
import jax.numpy as jnp
from jax import lax
from jax.experimental import pallas as pl
from jax.experimental.pallas import tpu as pltpu
from jax.experimental.pallas import tpu_sc as plsc

N_USER = 2000
N_ENT = 8000
N = 10000
K = 6
E_PER = 53334
D = 128
DR = 64

NC = 2
NS = 16
NW = NC * NS
EPW = 10240
E_PAD = NW * EPW
C = 128
N_PAD = 10240
RPS = N_PAD // NS
RB = 1000
NB = N // RB

_MESH = plsc.VectorSubcoreMesh(
    core_axis_name="c", subcore_axis_name="s", num_cores=NC, num_subcores=NS)


def _tables_body(ae_ref, te_ref, rel_ref, pt_ref):
    x = ae_ref[...]
    w = te_ref[0]
    p = jnp.dot(x, w, preferred_element_type=jnp.float32)
    r = rel_ref[pl.ds(pl.program_id(0), 1), :]
    t = jnp.tanh(p + r)
    pt_ref[0] = jnp.concatenate([p, t], axis=1)


def _tables(ae, transE, rel):
    return pl.pallas_call(
        _tables_body,
        grid=(K, NB),
        in_specs=[
            pl.BlockSpec((RB, D), lambda k, b: (b, 0)),
            pl.BlockSpec((1, D, DR), lambda k, b: (k, 0, 0)),
            pl.BlockSpec((K, DR), lambda k, b: (0, 0)),
        ],
        out_specs=pl.BlockSpec((1, RB, 2 * DR), lambda k, b: (k, b, 0)),
        out_shape=jax.ShapeDtypeStruct((K, N, 2 * DR), jnp.float32),
    )(ae, transE, rel)


_GPT = EPW // C


_N0D = 160


def _dots_body(pt_hbm, colp_hbm, rowt_hbm,
               part_hbm,
               colp_v, rowt_v, pbufs, tbufs, obs, psems, tsems, osems):
    c = lax.axis_index("c")
    s = lax.axis_index("s")
    cbase = pl.multiple_of(s * _N0D, 16)
    n_half = _N0D // 2

    @pl.when(c == 0)
    def _():
        pltpu.sync_copy(colp_hbm.at[pl.ds(cbase, _N0D)], colp_v)
        pltpu.sync_copy(rowt_hbm.at[pl.ds(cbase, _N0D)], rowt_v)

    def _issue(g, slot):
        pltpu.async_copy(pt_hbm.at[colp_v.at[g]], pbufs.at[slot], psems[slot])
        pltpu.async_copy(pt_hbm.at[rowt_v.at[g]], tbufs.at[slot], tsems[slot])

    def _wait(g, slot):
        pltpu.make_async_copy(pt_hbm.at[colp_v.at[g]], pbufs.at[slot],
                              psems[slot]).wait()
        pltpu.make_async_copy(pt_hbm.at[rowt_v.at[g]], tbufs.at[slot],
                              tsems[slot]).wait()

    def _compute(g, slot, first):
        _wait(g, slot)
        pbuf = pbufs.at[slot]
        tbuf = tbufs.at[slot]
        ob = obs.at[slot]

        @pl.when(jnp.logical_not(first))
        def _():
            pltpu.make_async_copy(
                ob, part_hbm.at[pl.ds(0, C >> 3)], osems[slot]).wait()

        @plsc.parallel_loop(0, C, unroll=8)
        def _e(e):
            acc = pbuf[e, pl.ds(0, 16)] * tbuf[e, pl.ds(DR, 16)]
            for j in range(1, DR // 16):
                acc = acc + (pbuf[e, pl.ds(j * 16, 16)]
                             * tbuf[e, pl.ds(DR + j * 16, 16)])
            ob[e >> 3, pl.ds((e & 7) * 16, 16)] = acc
        obase = pl.multiple_of((cbase + g) * (C >> 3), 16)
        pltpu.async_copy(ob, part_hbm.at[pl.ds(obase, C >> 3)], osems[slot])

    @pl.when(c == 0)
    def _():
        _issue(0, 0)

        def _pair(g2, _):
            g0 = 2 * g2
            _issue(g0 + 1, 1)
            _compute(g0, 0, g2 == 0)

            @pl.when(g2 < n_half - 1)
            def _():
                _issue(g0 + 2, 0)
            _compute(g0 + 1, 1, g2 == 0)
            return 0
        lax.fori_loop(0, n_half, _pair, 0)
        for slot in range(2):
            pltpu.make_async_copy(obs.at[slot], part_hbm.at[pl.ds(0, C >> 3)],
                                  osems[slot]).wait()


_dots = pl.kernel(
    _dots_body,
    out_type=jax.ShapeDtypeStruct((E_PAD // 8, 128), jnp.float32),
    mesh=_MESH,
    scratch_types=[
        pltpu.VMEM((_N0D, C), jnp.int32),
        pltpu.VMEM((_N0D, C), jnp.int32),
        pltpu.VMEM((2, C, 2 * DR), jnp.float32),
        pltpu.VMEM((2, C, 2 * DR), jnp.float32),
        pltpu.VMEM((2, C // 8, 128), jnp.float32),
        [pltpu.SemaphoreType.DMA] * 2,
        [pltpu.SemaphoreType.DMA] * 2,
        [pltpu.SemaphoreType.DMA] * 2,
    ],
)


_EB2 = 2048


def _ev_body(part_ref, ev_ref):
    x = part_ref[...]
    s = jnp.sum(x.reshape(_EB2 // 8, 8, 16), axis=2)
    ev_ref[...] = jnp.exp(s).reshape(_EB2 // 128, 128)


def _ev(part):
    return pl.pallas_call(
        _ev_body,
        grid=(E_PAD // _EB2,),
        in_specs=[pl.BlockSpec((_EB2 // 8, 128), lambda b: (b, 0))],
        out_specs=pl.BlockSpec((_EB2 // 128, 128), lambda b: (b, 0)),
        out_shape=jax.ShapeDtypeStruct((E_PAD // 128, 128), jnp.float32),
    )(part)


_CB3 = 2048


def _rsum_body(ev_hbm, rowd_hbm, rsum_hbm,
               ev2d, rowd2d, zbuf, acc_sh, sem):
    c = lax.axis_index("c")
    s = lax.axis_index("s")
    wid = s * NC + c

    def _z(i, _):
        zbuf[pl.ds(i * 16, 16)] = jnp.zeros((16,), jnp.float32)
        return 0
    lax.fori_loop(0, RPS // 16, _z, 0)
    pltpu.sync_copy(zbuf, acc_sh.at[pl.ds(s * RPS, RPS)])
    plsc.subcore_barrier()

    def _chunk(g, _):
        rbase = pl.multiple_of((wid * EPW + g * _CB3) // 128, 16)
        pltpu.sync_copy(ev_hbm.at[pl.ds(rbase, _CB3 // 128)], ev2d)
        pltpu.sync_copy(rowd_hbm.at[pl.ds(rbase, _CB3 // 128)], rowd2d)
        for j in range(_CB3 // 128):
            pltpu.sync_copy(ev2d.at[j], acc_sh.at[rowd2d.at[j]], add=True)
        return 0
    lax.fori_loop(0, EPW // _CB3, _chunk, 0)

    plsc.subcore_barrier()
    pltpu.sync_copy(acc_sh.at[pl.ds(s * RPS, RPS)], zbuf)
    pltpu.sync_copy(
        zbuf, rsum_hbm.at[pl.ds(pl.multiple_of(c * N_PAD + s * RPS, 128), RPS)])


_rsum = pl.kernel(
    _rsum_body,
    out_type=jax.ShapeDtypeStruct((NC * N_PAD,), jnp.float32),
    mesh=_MESH,
    scratch_types=[
        pltpu.VMEM((_CB3 // 128, 128), jnp.float32),
        pltpu.VMEM((_CB3 // 128, 128), jnp.int32),
        pltpu.VMEM((RPS,), jnp.float32),
        pltpu.VMEM_SHARED((N_PAD,), jnp.float32),
        pltpu.SemaphoreType.DMA,
    ],
)


def _inv_body(rs_ref, inv_ref):
    inv_ref[...] = 1.0 / jnp.sum(rs_ref[...], axis=0, keepdims=True)


def _inv(rsum):
    return pl.pallas_call(
        _inv_body,
        out_shape=jax.ShapeDtypeStruct((1, N_PAD), jnp.float32),
    )(rsum)


_N0A = 128
_N1A = 32


def _agg_body(x_hbm, ev_hbm, inv_hbm, col_hbm, rowd_hbm,
              neip_hbm,
              col_v, rowd_v, ev_v, invbufs, wbufs, rowsbufs,
              acc_sh, rsems, isems):
    c = lax.axis_index("c")
    s = lax.axis_index("s")
    cbase = pl.multiple_of(
        jnp.where(c == 0, s * _N0A, 16 * _N0A + s * _N1A), 8)
    n_segs = jnp.where(c == 0, _N0A // 8, _N1A // 8)

    rows0 = rowsbufs.at[0]

    def _zr(e, _):
        for j in range(D // 16):
            rows0[e, pl.ds(j * 16, 16)] = jnp.zeros((16,), jnp.float32)
        return 0
    lax.fori_loop(0, C, _zr, 0)
    for r in range(RPS // C):
        pltpu.sync_copy(rows0,
                        acc_sh.at[pl.ds(pl.multiple_of(s * RPS + r * C, 128), C)])
    plsc.subcore_barrier()

    def _issue(g, slot):
        pltpu.async_copy(x_hbm.at[col_v.at[g]], rowsbufs.at[slot],
                         rsems[slot])
        pltpu.async_copy(inv_hbm.at[rowd_v.at[g]], invbufs.at[slot],
                         isems[slot])

    def _compute(g, slot):
        pltpu.make_async_copy(x_hbm.at[col_v.at[g]], rowsbufs.at[slot],
                              rsems[slot]).wait()
        pltpu.make_async_copy(inv_hbm.at[rowd_v.at[g]], invbufs.at[slot],
                              isems[slot]).wait()
        rows_v = rowsbufs.at[slot]
        inv_v = invbufs.at[slot]
        w_v = wbufs.at[slot]

        @plsc.parallel_loop(0, C // 16, unroll=2)
        def _wv(i):
            sl = pl.ds(i * 16, 16)
            w_v[sl] = ev_v[g, sl] * inv_v[sl]

        @plsc.parallel_loop(0, C // 16, unroll=2)
        def _scale(eb):
            w16 = w_v[pl.ds(eb * 16, 16)]
            for l in range(16):
                e = eb * 16 + l
                w = w16[l]
                for j in range(D // 16):
                    sl = pl.ds(j * 16, 16)
                    rows_v[e, sl] = rows_v[e, sl] * w
        pltpu.sync_copy(rows_v, acc_sh.at[rowd_v.at[g]], add=True)

    def _seg(t, _):
        segbase = pl.multiple_of(cbase + t * 8, 8)
        pltpu.sync_copy(col_hbm.at[pl.ds(segbase, 8)], col_v)
        pltpu.sync_copy(rowd_hbm.at[pl.ds(segbase, 8)], rowd_v)
        pltpu.sync_copy(ev_hbm.at[pl.ds(segbase, 8)], ev_v)
        _issue(0, 0)
        for g in range(8):
            slot = g & 1
            if g < 7:
                _issue(g + 1, 1 - slot)
            _compute(g, slot)
        return 0
    lax.fori_loop(0, n_segs, _seg, 0)

    plsc.subcore_barrier()
    for r in range(RPS // C):
        off = pl.multiple_of(s * RPS + r * C, 128)
        pltpu.sync_copy(acc_sh.at[pl.ds(off, C)], rows0)
        pltpu.sync_copy(rows0, neip_hbm.at[c, pl.ds(off, C)])


_agg = pl.kernel(
    _agg_body,
    out_type=jax.ShapeDtypeStruct((NC, N_PAD, D), jnp.float32),
    mesh=_MESH,
    scratch_types=[
        pltpu.VMEM((8, C), jnp.int32),
        pltpu.VMEM((8, C), jnp.int32),
        pltpu.VMEM((8, C), jnp.float32),
        pltpu.VMEM((2, C), jnp.float32),
        pltpu.VMEM((2, C), jnp.float32),
        pltpu.VMEM((2, C, D), jnp.float32),
        pltpu.VMEM_SHARED((N_PAD, D), jnp.float32),
        [pltpu.SemaphoreType.DMA] * 2,
        [pltpu.SemaphoreType.DMA] * 2,
    ],
)


def _layer_body(neip_ref, x_ref, w1_ref, b1_ref, w2_ref, b2_ref,
                xn_ref, y_ref):
    nei = neip_ref[0] + neip_ref[1]
    x = x_ref[...]
    w1 = w1_ref[...] + b1_ref[...]
    w2 = w2_ref[...] + b2_ref[...]
    sm = jnp.dot(nei + x, w1, preferred_element_type=jnp.float32)
    sm = jnp.where(sm >= 0, sm, 0.2 * sm)
    bi = jnp.dot(nei * x, w2, preferred_element_type=jnp.float32)
    bi = jnp.where(bi >= 0, bi, 0.2 * bi)
    xn = sm + bi
    xn_ref[...] = xn
    nrm = jnp.sqrt(jnp.sum(xn * xn, axis=1, keepdims=True))
    y_ref[...] = xn / jnp.maximum(nrm, 1e-12)


def _layer(neip, x, W1, b1, W2, b2):
    return pl.pallas_call(
        _layer_body,
        grid=(NB,),
        in_specs=[
            pl.BlockSpec((NC, RB, D), lambda b: (0, b, 0)),
            pl.BlockSpec((RB, D), lambda b: (b, 0)),
            pl.BlockSpec((D, D), lambda b: (0, 0)),
            pl.BlockSpec((1, D), lambda b: (0, 0)),
            pl.BlockSpec((D, D), lambda b: (0, 0)),
            pl.BlockSpec((1, D), lambda b: (0, 0)),
        ],
        out_specs=[
            pl.BlockSpec((RB, D), lambda b: (b, 0)),
            pl.BlockSpec((RB, D), lambda b: (b, 0)),
        ],
        out_shape=[
            jax.ShapeDtypeStruct((N, D), jnp.float32),
            jax.ShapeDtypeStruct((N, D), jnp.float32),
        ],
    )(neip, x, W1, b1, W2, b2)


def kernel(user_embed, entity_embed, relation_embed, transE,
           W1_0, b1_0, W2_0, b2_0, W1_1, b1_1, W2_1, b2_1,
           edge_index):
    ae = jnp.concatenate([user_embed, entity_embed], axis=0)
    PT = _tables(ae, transE, relation_embed)
    PTf = PT.reshape(K * N, 2 * DR)

    kk = jnp.arange(K, dtype=jnp.int32)[:, None] * N
    row = edge_index[:, :, 0]
    col = edge_index[:, :, 1]
    pad = E_PAD - K * E_PER
    zpad = jnp.zeros((pad,), jnp.int32)
    rowt = jnp.concatenate([(row + kk).reshape(-1), zpad])
    colp = jnp.concatenate([(col + kk).reshape(-1), zpad])
    colg = jnp.concatenate([col.reshape(-1), zpad])
    rowd = jnp.concatenate([row.reshape(-1), jnp.full((pad,), N, jnp.int32)])

    part = _dots(PTf, colp.reshape(E_PAD // C, C), rowt.reshape(E_PAD // C, C))
    ev2d = _ev(part)
    rowd2d = rowd.reshape(E_PAD // 128, 128)
    colg2d = colg.reshape(E_PAD // 128, 128)
    rsum = _rsum(ev2d, rowd2d).reshape(NC, N_PAD)
    inv = _inv(rsum).reshape(N_PAD)

    outs = [ae]
    x = ae
    for (W1, b1, W2, b2) in ((W1_0, b1_0, W2_0, b2_0),
                             (W1_1, b1_1, W2_1, b2_1)):
        neip = _agg(x, ev2d, inv, colg2d, rowd2d)
        x, y = _layer(neip, x, W1, b1, W2, b2)
        outs.append(y)

    out = jnp.concatenate(outs, axis=1)
    return out[:N_USER], out[N_USER:]


# speedup vs baseline: 1.0819x; 1.0819x over previous
import functools

import jax
import jax.numpy as jnp
from jax import lax
from jax.experimental import pallas as pl
from jax.experimental.pallas import tpu as pltpu
from jax.experimental.pallas import tpu_sc as plsc

N_USER = 2000
N_ENT = 8000
N = 10000
K = 6
E_PER = 53334
D = 128
DR = 64

NC = 2
NS = 16
NW = NC * NS
EPW = 10240
E_PAD = NW * EPW
C = 128
N_PAD = 10240
RPS = N_PAD // NS
RB = 1000
NB = N // RB

_MESH = plsc.VectorSubcoreMesh(
    core_axis_name="c", subcore_axis_name="s", num_cores=NC, num_subcores=NS)


def _tables_body(ae_ref, te_ref, rel_ref, pt_ref):
    x = ae_ref[...]
    w = te_ref[0]
    p = jnp.dot(x, w, preferred_element_type=jnp.float32)
    r = rel_ref[pl.ds(pl.program_id(0), 1), :]
    t = jnp.tanh(p + r)
    pt_ref[0] = jnp.concatenate([p, t], axis=1)


def _tables(ae, transE, rel):
    return pl.pallas_call(
        _tables_body,
        grid=(K, NB),
        in_specs=[
            pl.BlockSpec((RB, D), lambda k, b: (b, 0)),
            pl.BlockSpec((1, D, DR), lambda k, b: (k, 0, 0)),
            pl.BlockSpec((K, DR), lambda k, b: (0, 0)),
        ],
        out_specs=pl.BlockSpec((1, RB, 2 * DR), lambda k, b: (k, b, 0)),
        out_shape=jax.ShapeDtypeStruct((K, N, 2 * DR), jnp.float32),
    )(ae, transE, rel)


_GPT = EPW // C


def _dots_body(pt_hbm, colp_hbm, rowt_hbm,
               part_hbm,
               colp_v, rowt_v, pbufs, tbufs, obs, psems, tsems, osems):
    c = lax.axis_index("c")
    s = lax.axis_index("s")
    wid = s * NC + c
    cbase = pl.multiple_of(wid * _GPT, 16)
    n_half = _GPT // 2
    pltpu.sync_copy(colp_hbm.at[pl.ds(cbase, _GPT)], colp_v)
    pltpu.sync_copy(rowt_hbm.at[pl.ds(cbase, _GPT)], rowt_v)

    def _issue(g, slot):
        pltpu.async_copy(pt_hbm.at[colp_v.at[g]], pbufs.at[slot], psems[slot])
        pltpu.async_copy(pt_hbm.at[rowt_v.at[g]], tbufs.at[slot], tsems[slot])

    def _wait(g, slot):
        pltpu.make_async_copy(pt_hbm.at[colp_v.at[g]], pbufs.at[slot],
                              psems[slot]).wait()
        pltpu.make_async_copy(pt_hbm.at[rowt_v.at[g]], tbufs.at[slot],
                              tsems[slot]).wait()

    def _compute(g, slot, first):
        _wait(g, slot)
        pbuf = pbufs.at[slot]
        tbuf = tbufs.at[slot]
        ob = obs.at[slot]

        @pl.when(jnp.logical_not(first))
        def _():
            pltpu.make_async_copy(
                ob, part_hbm.at[pl.ds(0, C >> 3)], osems[slot]).wait()

        @plsc.parallel_loop(0, C, unroll=8)
        def _e(e):
            acc = pbuf[e, pl.ds(0, 16)] * tbuf[e, pl.ds(DR, 16)]
            for j in range(1, DR // 16):
                acc = acc + (pbuf[e, pl.ds(j * 16, 16)]
                             * tbuf[e, pl.ds(DR + j * 16, 16)])
            ob[e >> 3, pl.ds((e & 7) * 16, 16)] = acc
        obase = pl.multiple_of((cbase + g) * (C >> 3), 16)
        pltpu.async_copy(ob, part_hbm.at[pl.ds(obase, C >> 3)], osems[slot])

    _issue(0, 0)

    def _pair(g2, _):
        g0 = 2 * g2
        _issue(g0 + 1, 1)
        _compute(g0, 0, g2 == 0)

        @pl.when(g2 < n_half - 1)
        def _():
            _issue(g0 + 2, 0)
        _compute(g0 + 1, 1, g2 == 0)
        return 0
    lax.fori_loop(0, n_half, _pair, 0)
    for slot in range(2):
        pltpu.make_async_copy(obs.at[slot], part_hbm.at[pl.ds(0, C >> 3)],
                              osems[slot]).wait()


_dots = pl.kernel(
    _dots_body,
    out_type=jax.ShapeDtypeStruct((E_PAD // 8, 128), jnp.float32),
    mesh=_MESH,
    scratch_types=[
        pltpu.VMEM((_GPT, C), jnp.int32),
        pltpu.VMEM((_GPT, C), jnp.int32),
        pltpu.VMEM((2, C, 2 * DR), jnp.float32),
        pltpu.VMEM((2, C, 2 * DR), jnp.float32),
        pltpu.VMEM((2, C // 8, 128), jnp.float32),
        [pltpu.SemaphoreType.DMA] * 2,
        [pltpu.SemaphoreType.DMA] * 2,
        [pltpu.SemaphoreType.DMA] * 2,
    ],
)


_EB2 = 2048


def _ev_body(part_ref, ev_ref):
    x = part_ref[...]
    s = jnp.sum(x.reshape(_EB2 // 8, 8, 16), axis=2)
    ev_ref[...] = jnp.exp(s).reshape(_EB2 // 128, 128)


def _ev(part):
    return pl.pallas_call(
        _ev_body,
        grid=(E_PAD // _EB2,),
        in_specs=[pl.BlockSpec((_EB2 // 8, 128), lambda b: (b, 0))],
        out_specs=pl.BlockSpec((_EB2 // 128, 128), lambda b: (b, 0)),
        out_shape=jax.ShapeDtypeStruct((E_PAD // 128, 128), jnp.float32),
    )(part)


_CB3 = 2048


def _rsum_body(ev_hbm, rowd_hbm, rsum_hbm,
               ev2d, rowd2d, zbuf, acc_sh, sem):
    c = lax.axis_index("c")
    s = lax.axis_index("s")
    wid = s * NC + c

    def _z(i, _):
        zbuf[pl.ds(i * 16, 16)] = jnp.zeros((16,), jnp.float32)
        return 0
    lax.fori_loop(0, RPS // 16, _z, 0)
    pltpu.sync_copy(zbuf, acc_sh.at[pl.ds(s * RPS, RPS)])
    plsc.subcore_barrier()

    def _chunk(g, _):
        rbase = pl.multiple_of((wid * EPW + g * _CB3) // 128, 16)
        pltpu.sync_copy(ev_hbm.at[pl.ds(rbase, _CB3 // 128)], ev2d)
        pltpu.sync_copy(rowd_hbm.at[pl.ds(rbase, _CB3 // 128)], rowd2d)
        for j in range(_CB3 // 128):
            pltpu.sync_copy(ev2d.at[j], acc_sh.at[rowd2d.at[j]], add=True)
        return 0
    lax.fori_loop(0, EPW // _CB3, _chunk, 0)

    plsc.subcore_barrier()
    pltpu.sync_copy(acc_sh.at[pl.ds(s * RPS, RPS)], zbuf)
    pltpu.sync_copy(
        zbuf, rsum_hbm.at[pl.ds(pl.multiple_of(c * N_PAD + s * RPS, 128), RPS)])


_rsum = pl.kernel(
    _rsum_body,
    out_type=jax.ShapeDtypeStruct((NC * N_PAD,), jnp.float32),
    mesh=_MESH,
    scratch_types=[
        pltpu.VMEM((_CB3 // 128, 128), jnp.float32),
        pltpu.VMEM((_CB3 // 128, 128), jnp.int32),
        pltpu.VMEM((RPS,), jnp.float32),
        pltpu.VMEM_SHARED((N_PAD,), jnp.float32),
        pltpu.SemaphoreType.DMA,
    ],
)


def _inv_body(rs_ref, inv_ref):
    inv_ref[...] = 1.0 / jnp.sum(rs_ref[...], axis=0, keepdims=True)


def _inv(rsum):
    return pl.pallas_call(
        _inv_body,
        out_shape=jax.ShapeDtypeStruct((1, N_PAD), jnp.float32),
    )(rsum)


_N0A = 128
_N1A = 32


def _agg_body(x_hbm, ev_hbm, inv_hbm, col_hbm, rowd_hbm,
              neip_hbm,
              col_v, rowd_v, ev_v, invbufs, wbufs, rowsbufs,
              acc_sh, rsems, isems):
    c = lax.axis_index("c")
    s = lax.axis_index("s")
    cbase = pl.multiple_of(
        jnp.where(c == 0, s * _N0A, 16 * _N0A + s * _N1A), 8)
    n_segs = jnp.where(c == 0, _N0A // 8, _N1A // 8)

    rows0 = rowsbufs.at[0]

    def _zr(e, _):
        for j in range(D // 16):
            rows0[e, pl.ds(j * 16, 16)] = jnp.zeros((16,), jnp.float32)
        return 0
    lax.fori_loop(0, C, _zr, 0)
    for r in range(RPS // C):
        pltpu.sync_copy(rows0,
                        acc_sh.at[pl.ds(pl.multiple_of(s * RPS + r * C, 128), C)])
    plsc.subcore_barrier()

    def _issue(g, slot):
        pltpu.async_copy(x_hbm.at[col_v.at[g]], rowsbufs.at[slot],
                         rsems[slot])
        pltpu.async_copy(inv_hbm.at[rowd_v.at[g]], invbufs.at[slot],
                         isems[slot])

    def _compute(g, slot):
        pltpu.make_async_copy(x_hbm.at[col_v.at[g]], rowsbufs.at[slot],
                              rsems[slot]).wait()
        pltpu.make_async_copy(inv_hbm.at[rowd_v.at[g]], invbufs.at[slot],
                              isems[slot]).wait()
        rows_v = rowsbufs.at[slot]
        inv_v = invbufs.at[slot]
        w_v = wbufs.at[slot]

        @plsc.parallel_loop(0, C // 16, unroll=2)
        def _wv(i):
            sl = pl.ds(i * 16, 16)
            w_v[sl] = ev_v[g, sl] * inv_v[sl]

        @plsc.parallel_loop(0, C // 16, unroll=2)
        def _scale(eb):
            w16 = w_v[pl.ds(eb * 16, 16)]
            for l in range(16):
                e = eb * 16 + l
                w = w16[l]
                for j in range(D // 16):
                    sl = pl.ds(j * 16, 16)
                    rows_v[e, sl] = rows_v[e, sl] * w
        pltpu.sync_copy(rows_v, acc_sh.at[rowd_v.at[g]], add=True)

    def _seg(t, _):
        segbase = pl.multiple_of(cbase + t * 8, 8)
        pltpu.sync_copy(col_hbm.at[pl.ds(segbase, 8)], col_v)
        pltpu.sync_copy(rowd_hbm.at[pl.ds(segbase, 8)], rowd_v)
        pltpu.sync_copy(ev_hbm.at[pl.ds(segbase, 8)], ev_v)
        _issue(0, 0)
        for g in range(8):
            slot = g & 1
            if g < 7:
                _issue(g + 1, 1 - slot)
            _compute(g, slot)
        return 0
    lax.fori_loop(0, n_segs, _seg, 0)

    plsc.subcore_barrier()
    for r in range(RPS // C):
        off = pl.multiple_of(s * RPS + r * C, 128)
        pltpu.sync_copy(acc_sh.at[pl.ds(off, C)], rows0)
        pltpu.sync_copy(rows0, neip_hbm.at[c, pl.ds(off, C)])


_agg = pl.kernel(
    _agg_body,
    out_type=jax.ShapeDtypeStruct((NC, N_PAD, D), jnp.float32),
    mesh=_MESH,
    scratch_types=[
        pltpu.VMEM((8, C), jnp.int32),
        pltpu.VMEM((8, C), jnp.int32),
        pltpu.VMEM((8, C), jnp.float32),
        pltpu.VMEM((2, C), jnp.float32),
        pltpu.VMEM((2, C), jnp.float32),
        pltpu.VMEM((2, C, D), jnp.float32),
        pltpu.VMEM_SHARED((N_PAD, D), jnp.float32),
        [pltpu.SemaphoreType.DMA] * 2,
        [pltpu.SemaphoreType.DMA] * 2,
    ],
)


def _layer_body(neip_ref, x_ref, w1_ref, b1_ref, w2_ref, b2_ref,
                xn_ref, y_ref):
    nei = neip_ref[0] + neip_ref[1]
    x = x_ref[...]
    w1 = w1_ref[...] + b1_ref[...]
    w2 = w2_ref[...] + b2_ref[...]
    sm = jnp.dot(nei + x, w1, preferred_element_type=jnp.float32)
    sm = jnp.where(sm >= 0, sm, 0.2 * sm)
    bi = jnp.dot(nei * x, w2, preferred_element_type=jnp.float32)
    bi = jnp.where(bi >= 0, bi, 0.2 * bi)
    xn = sm + bi
    xn_ref[...] = xn
    nrm = jnp.sqrt(jnp.sum(xn * xn, axis=1, keepdims=True))
    y_ref[...] = xn / jnp.maximum(nrm, 1e-12)


def _layer(neip, x, W1, b1, W2, b2):
    return pl.pallas_call(
        _layer_body,
        grid=(NB,),
        in_specs=[
            pl.BlockSpec((NC, RB, D), lambda b: (0, b, 0)),
            pl.BlockSpec((RB, D), lambda b: (b, 0)),
            pl.BlockSpec((D, D), lambda b: (0, 0)),
            pl.BlockSpec((1, D), lambda b: (0, 0)),
            pl.BlockSpec((D, D), lambda b: (0, 0)),
            pl.BlockSpec((1, D), lambda b: (0, 0)),
        ],
        out_specs=[
            pl.BlockSpec((RB, D), lambda b: (b, 0)),
            pl.BlockSpec((RB, D), lambda b: (b, 0)),
        ],
        out_shape=[
            jax.ShapeDtypeStruct((N, D), jnp.float32),
            jax.ShapeDtypeStruct((N, D), jnp.float32),
        ],
    )(neip, x, W1, b1, W2, b2)


def kernel(user_embed, entity_embed, relation_embed, transE,
           W1_0, b1_0, W2_0, b2_0, W1_1, b1_1, W2_1, b2_1,
           edge_index):
    ae = jnp.concatenate([user_embed, entity_embed], axis=0)
    PT = _tables(ae, transE, relation_embed)
    PTf = PT.reshape(K * N, 2 * DR)

    kk = jnp.arange(K, dtype=jnp.int32)[:, None] * N
    row = edge_index[:, :, 0]
    col = edge_index[:, :, 1]
    pad = E_PAD - K * E_PER
    zpad = jnp.zeros((pad,), jnp.int32)
    rowt = jnp.concatenate([(row + kk).reshape(-1), zpad])
    colp = jnp.concatenate([(col + kk).reshape(-1), zpad])
    colg = jnp.concatenate([col.reshape(-1), zpad])
    rowd = jnp.concatenate([row.reshape(-1), jnp.full((pad,), N, jnp.int32)])

    part = _dots(PTf, colp.reshape(E_PAD // C, C), rowt.reshape(E_PAD // C, C))
    ev2d = _ev(part)
    rowd2d = rowd.reshape(E_PAD // 128, 128)
    colg2d = colg.reshape(E_PAD // 128, 128)
    rsum = _rsum(ev2d, rowd2d).reshape(NC, N_PAD)
    inv = _inv(rsum).reshape(N_PAD)

    outs = [ae]
    x = ae
    for (W1, b1, W2, b2) in ((W1_0, b1_0, W2_0, b2_0),
                             (W1_1, b1_1, W2_1, b2_1)):
        neip = _agg(x, ev2d, inv, colg2d, rowd2d)
        x, y = _layer(neip, x, W1, b1, W2, b2)
        outs.append(y)

    out = jnp.concatenate(outs, axis=1)
    return out[:N_USER], out[N_USER:]
